# baseline (speedup 1.0000x reference)
import functools

import jax
import jax.numpy as jnp
from jax import lax
from jax.experimental import pallas as pl
from jax.experimental.pallas import tpu as pltpu
from jax.experimental.pallas import tpu_sc as plsc

NODES = 10000
EDGES = 320000
ELABEL = 64000
D = 128

NC = 2
NS = 16
NW = NC * NS

EPT = EDGES // NW
K = 80
KA = 40
NCHUNK = EPT // KA
NB = 5
LPT = ELABEL // NW
LCHUNK = LPT // K
BR = 80
NBLK = NODES // BR

_MESH = plsc.VectorSubcoreMesh(
    core_axis_name="c", subcore_axis_name="s", num_cores=NC, num_subcores=NS)


def _fill2d(ref, rows, cols, val):
  def row(i, _):
    for k in range(cols // 16):
      ref[i, pl.ds(k * 16, 16)] = jnp.full((16,), val, jnp.float32)
    return 0
  lax.fori_loop(0, rows, row, 0)


NBR = 6

_DEG_SCRATCH = [
    pltpu.VMEM((NCHUNK, KA), jnp.int32),
    pltpu.VMEM((KA, 16), jnp.float32),
    pltpu.VMEM((BR, 16), jnp.float32),
    pltpu.VMEM_SHARED((NODES, 16), jnp.float32),
    pltpu.SemaphoreType.DMA,
]


@functools.partial(
    pl.kernel,
    out_type=jax.ShapeDtypeStruct((NC, NODES, 16), jnp.float32),
    mesh=_MESH,
    scratch_types=_DEG_SCRATCH,
    compiler_params=pltpu.CompilerParams(use_tc_tiling_on_sc=False))
def _deg(dst_hbm, deg_out, idxd_all, ones_v, zd_v, deg_s, dsem):
  cid = lax.axis_index("c")
  sid = lax.axis_index("s")
  wid = sid * NC + cid

  pltpu.sync_copy(dst_hbm.at[wid], idxd_all)
  _fill2d(ones_v, KA, 16, 1.0)
  _fill2d(zd_v, BR, 16, 0.0)

  def zcp(j, _):
    b = sid + j * NS

    @pl.when(b < NBLK)
    def _():
      pltpu.sync_copy(zd_v, deg_s.at[pl.ds(b * BR, BR)])
    return 0
  lax.fori_loop(0, (NBLK + NS - 1) // NS, zcp, 0)
  plsc.subcore_barrier()

  def fire(c, _):
    pltpu.async_copy(ones_v, deg_s.at[idxd_all.at[c]], dsem, add=True)
    return 0
  lax.fori_loop(0, NCHUNK, fire, 0)

  def drain(c, _):
    pltpu.make_async_copy(ones_v, deg_s.at[idxd_all.at[0]], dsem).wait()
    return 0
  lax.fori_loop(0, NCHUNK, drain, 0)
  plsc.subcore_barrier()

  def wout(j, _):
    b = sid + j * NS

    @pl.when(b < NBLK)
    def _():
      r0 = b * BR
      pltpu.sync_copy(deg_s.at[pl.ds(r0, BR)], deg_out.at[cid, pl.ds(r0, BR)])
    return 0
  lax.fori_loop(0, (NBLK + NS - 1) // NS, wout, 0)


_AGG_SCRATCH = [
    pltpu.VMEM((NCHUNK, KA), jnp.int32),
    pltpu.VMEM((NCHUNK, KA), jnp.int32),
    pltpu.VMEM_SHARED((NODES, D), jnp.float32),
]
_AGG_SCRATCH += [pltpu.VMEM((KA, D), jnp.float32) for _ in range(NBR)]
_AGG_SCRATCH += [pltpu.SemaphoreType.DMA for _ in range(2 * NBR)]


@functools.partial(
    pl.kernel,
    out_type=jax.ShapeDtypeStruct((NC, NODES, D), jnp.float32),
    mesh=_MESH,
    scratch_types=_AGG_SCRATCH,
    compiler_params=pltpu.CompilerParams(use_tc_tiling_on_sc=False))
def _agg(x_hbm, src_hbm, dst_hbm, zero_hbm, agg_out, idxs_all, idxd_all,
         agg_s, *bufs):
  rows_v = bufs[:NBR]
  gsem = bufs[NBR:2 * NBR]
  ssem = bufs[2 * NBR:3 * NBR]
  cid = lax.axis_index("c")
  sid = lax.axis_index("s")
  wid = sid * NC + cid

  pltpu.sync_copy(src_hbm.at[wid], idxs_all)
  pltpu.sync_copy(dst_hbm.at[wid], idxd_all)

  def zcp(j, _):
    b = sid + j * NS

    @pl.when(b < NBLK)
    def _():
      pltpu.sync_copy(zero_hbm.at[pl.ds(b * BR, BR)],
                      agg_s.at[pl.ds(b * BR, BR)])
    return 0
  lax.fori_loop(0, (NBLK + NS - 1) // NS, zcp, 0)
  plsc.subcore_barrier()

  def g_start(c, b):
    pltpu.async_copy(x_hbm.at[idxs_all.at[c]], rows_v[b], gsem[b])

  def g_wait(b):
    pltpu.make_async_copy(x_hbm.at[idxs_all.at[0]], rows_v[b], gsem[b]).wait()

  def s_start(c, b):
    pltpu.async_copy(rows_v[b], agg_s.at[idxd_all.at[c]], ssem[b], add=True)

  def s_wait(b):
    pltpu.make_async_copy(rows_v[b], agg_s.at[idxd_all.at[0]], ssem[b]).wait()

  for b in range(NBR):
    g_start(b, b)

  def outer(i, _):
    g = i * NBR
    for b in range(NBR):
      c = g + b
      g_wait(b)
      s_start(c, b)
      s_wait(b)

      @pl.when(c + NBR < NCHUNK)
      def _():
        g_start(c + NBR, b)
    return 0
  lax.fori_loop(0, NCHUNK // NBR, outer, 0)

  for t in range(NCHUNK - (NCHUNK // NBR) * NBR):
    c = (NCHUNK // NBR) * NBR + t
    b = c % NBR
    g_wait(b)
    s_start(c, b)
    s_wait(b)
  plsc.subcore_barrier()

  def wout(j, _):
    b = sid + j * NS

    @pl.when(b < NBLK)
    def _():
      r0 = b * BR
      pltpu.sync_copy(agg_s.at[pl.ds(r0, BR)], agg_out.at[cid, pl.ds(r0, BR)])
    return 0
  lax.fori_loop(0, (NBLK + NS - 1) // NS, wout, 0)


DNB = 2
ZREP = 4

_DEC_SCRATCH = [
    pltpu.VMEM((LCHUNK, K), jnp.int32),
    pltpu.VMEM((LCHUNK, K), jnp.int32),
    pltpu.VMEM((LPT,), jnp.float32),
]
_DEC_SCRATCH += [pltpu.VMEM((K, D), jnp.float32) for _ in range(2 * DNB)]
_DEC_SCRATCH += [pltpu.SemaphoreType.DMA for _ in range(DNB)]


@functools.partial(
    pl.kernel,
    out_type=jax.ShapeDtypeStruct((ELABEL,), jnp.float32),
    mesh=_MESH,
    scratch_types=_DEC_SCRATCH,
    compiler_params=pltpu.CompilerParams(use_tc_tiling_on_sc=False))
def _decode(z_hbm, li_hbm, lj_hbm, out_hbm, idxi_all, idxj_all, out_v, *bufs):
  zi_v = bufs[:DNB]
  zj_v = bufs[DNB:2 * DNB]
  gsem = bufs[2 * DNB:]
  cid = lax.axis_index("c")
  sid = lax.axis_index("s")
  wid = sid * NC + cid

  pltpu.sync_copy(li_hbm.at[wid], idxi_all)
  pltpu.sync_copy(lj_hbm.at[wid], idxj_all)

  roff = (wid % ZREP) * NODES

  def shift(i, _):
    for kk in range(K // 16):
      sl = pl.ds(kk * 16, 16)
      off = jnp.full((16,), 1, jnp.int32) * roff
      idxi_all[i, sl] = idxi_all[i, sl] + off
      idxj_all[i, sl] = idxj_all[i, sl] + off
    return 0
  lax.fori_loop(0, LCHUNK, shift, 0)

  lanes = lax.iota(jnp.int32, 16)

  def g_start(c, b):
    pltpu.async_copy(z_hbm.at[idxi_all.at[c]], zi_v[b], gsem[b])
    pltpu.async_copy(z_hbm.at[idxj_all.at[c]], zj_v[b], gsem[b])

  def g_wait(b):
    pltpu.make_async_copy(z_hbm.at[idxi_all.at[0]], zi_v[b], gsem[b]).wait()
    pltpu.make_async_copy(z_hbm.at[idxj_all.at[0]], zj_v[b], gsem[b]).wait()

  for b in range(DNB):
    g_start(b, b)

  def comb(u, v, d, mask):
    perm = jnp.bitwise_xor(lanes, d)
    uu = u + jnp.take_along_axis(u, perm, axis=0)
    vv = v + jnp.take_along_axis(v, perm, axis=0)
    return jnp.where(mask, vv, uu)

  masks = {d: (lanes & d) != 0 for d in (1, 2, 4, 8)}

  def compute_chunk(c, b):
    for g16 in range(K // 16):
      row0 = g16 * 16
      t = []
      for rr in range(16):
        r = row0 + rr
        acc = zi_v[b][r, pl.ds(0, 16)] * zj_v[b][r, pl.ds(0, 16)]
        for kk in range(1, D // 16):
          sl = pl.ds(kk * 16, 16)
          acc = acc + zi_v[b][r, sl] * zj_v[b][r, sl]
        t.append(acc)
      for d in (1, 2, 4, 8):
        t = [comb(t[2 * j], t[2 * j + 1], d, masks[d])
             for j in range(len(t) // 2)]
      out_v[pl.ds(c * K + row0, 16)] = t[0]

  def outer(i, _):
    g0 = i * DNB
    for b in range(DNB):
      c = g0 + b
      g_wait(b)
      compute_chunk(c, b)

      @pl.when(c + DNB < LCHUNK)
      def _():
        g_start(c + DNB, b)
    return 0
  lax.fori_loop(0, (LCHUNK - 1) // DNB, outer, 0)

  g_wait((LCHUNK - 1) % DNB)
  compute_chunk(LCHUNK - 1, (LCHUNK - 1) % DNB)

  pltpu.sync_copy(out_v, out_hbm.at[pl.ds(wid * LPT, LPT)])


def _tc_layer(p, pd, feat, Wl, b, Wr, relu, rep=1):
  R = 1000

  def body(p_ref, pd_ref, f_ref, wl_ref, b_ref, wr_ref, o_ref):
    agg = p_ref[0] + p_ref[1]
    deg = pd_ref[0, :, :1] + pd_ref[1, :, :1]
    mean = agg / jnp.maximum(deg, 1.0)
    acc = lax.dot_general(mean, wl_ref[...], (((1,), (1,)), ((), ())),
                          preferred_element_type=jnp.float32)
    acc = acc + b_ref[...]
    acc = acc + lax.dot_general(f_ref[...], wr_ref[...],
                                (((1,), (1,)), ((), ())),
                                preferred_element_type=jnp.float32)
    if relu:
      acc = jnp.maximum(acc, 0.0)
    if rep == 1:
      o_ref[...] = acc
    else:
      o_ref[...] = jnp.broadcast_to(acc[None], (rep, R, D))

  out_specs = (pl.BlockSpec((R, D), lambda i: (i, 0)) if rep == 1 else
               pl.BlockSpec((rep, R, D), lambda i: (0, i, 0)))
  out_shape = (jax.ShapeDtypeStruct((NODES, D), jnp.float32) if rep == 1 else
               jax.ShapeDtypeStruct((rep, NODES, D), jnp.float32))
  return pl.pallas_call(
      body,
      grid=(NODES // R,),
      in_specs=[
          pl.BlockSpec((NC, R, D), lambda i: (0, i, 0)),
          pl.BlockSpec((NC, R, 16), lambda i: (0, i, 0)),
          pl.BlockSpec((R, D), lambda i: (i, 0)),
          pl.BlockSpec((D, D), lambda i: (0, 0)),
          pl.BlockSpec((1, D), lambda i: (0, 0)),
          pl.BlockSpec((D, D), lambda i: (0, 0)),
      ],
      out_specs=out_specs,
      out_shape=out_shape,
  )(p, pd, feat, Wl, b.reshape(1, D), Wr)


def kernel(x, edge_index, edge_label_idx, W1l, b1, W1r, W2l, b2, W2r):
  src = edge_index[0].reshape(NW, NCHUNK, KA)
  dst = edge_index[1].reshape(NW, NCHUNK, KA)
  li = edge_label_idx[0].reshape(NW, LCHUNK, K)
  lj = edge_label_idx[1].reshape(NW, LCHUNK, K)

  zeros = jnp.zeros((NODES, D), jnp.float32)
  pd = _deg(dst)
  p1 = _agg(x, src, dst, zeros)
  h = _tc_layer(p1, pd, x, W1l, b1, W1r, relu=True)
  p2 = _agg(h, src, dst, zeros)
  z = _tc_layer(p2, pd, h, W2l, b2, W2r, relu=False, rep=ZREP)
  return _decode(z.reshape(ZREP * NODES, D), li, lj)


# speedup vs baseline: 1.0532x; 1.0532x over previous
import functools

import jax
import jax.numpy as jnp
from jax import lax
from jax.experimental import pallas as pl
from jax.experimental.pallas import tpu as pltpu
from jax.experimental.pallas import tpu_sc as plsc

NODES = 10000
EDGES = 320000
ELABEL = 64000
D = 128

NC = 2
NS = 16
NW = NC * NS

EPT = EDGES // NW
K = 80
KA = 40
NCHUNK = EPT // KA
NB = 5
LPT = ELABEL // NW
LCHUNK = LPT // K
BR = 80
NBLK = NODES // BR

_MESH = plsc.VectorSubcoreMesh(
    core_axis_name="c", subcore_axis_name="s", num_cores=NC, num_subcores=NS)


def _fill2d(ref, rows, cols, val):
  def row(i, _):
    for k in range(cols // 16):
      ref[i, pl.ds(k * 16, 16)] = jnp.full((16,), val, jnp.float32)
    return 0
  lax.fori_loop(0, rows, row, 0)


NBR = 6

_DEG_SCRATCH = [
    pltpu.VMEM((NCHUNK, KA), jnp.int32),
    pltpu.VMEM((KA, 16), jnp.float32),
    pltpu.VMEM((BR, 16), jnp.float32),
    pltpu.VMEM_SHARED((NODES, 16), jnp.float32),
    pltpu.SemaphoreType.DMA,
]


@functools.partial(
    pl.kernel,
    out_type=jax.ShapeDtypeStruct((NC, NODES, 16), jnp.float32),
    mesh=_MESH,
    scratch_types=_DEG_SCRATCH,
    compiler_params=pltpu.CompilerParams(use_tc_tiling_on_sc=False))
def _deg(dst_hbm, deg_out, idxd_all, ones_v, zd_v, deg_s, dsem):
  cid = lax.axis_index("c")
  sid = lax.axis_index("s")
  wid = sid * NC + cid

  pltpu.sync_copy(dst_hbm.at[wid], idxd_all)
  _fill2d(ones_v, KA, 16, 1.0)
  _fill2d(zd_v, BR, 16, 0.0)

  def zcp(j, _):
    b = sid + j * NS

    @pl.when(b < NBLK)
    def _():
      pltpu.sync_copy(zd_v, deg_s.at[pl.ds(b * BR, BR)])
    return 0
  lax.fori_loop(0, (NBLK + NS - 1) // NS, zcp, 0)
  plsc.subcore_barrier()

  def fire(c, _):
    pltpu.async_copy(ones_v, deg_s.at[idxd_all.at[c]], dsem, add=True)
    return 0
  lax.fori_loop(0, NCHUNK, fire, 0)

  def drain(c, _):
    pltpu.make_async_copy(ones_v, deg_s.at[idxd_all.at[0]], dsem).wait()
    return 0
  lax.fori_loop(0, NCHUNK, drain, 0)
  plsc.subcore_barrier()

  def wout(j, _):
    b = sid + j * NS

    @pl.when(b < NBLK)
    def _():
      r0 = b * BR
      pltpu.sync_copy(deg_s.at[pl.ds(r0, BR)], deg_out.at[cid, pl.ds(r0, BR)])
    return 0
  lax.fori_loop(0, (NBLK + NS - 1) // NS, wout, 0)


_AGG_SCRATCH = [
    pltpu.VMEM((NCHUNK, KA), jnp.int32),
    pltpu.VMEM((NCHUNK, KA), jnp.int32),
    pltpu.VMEM_SHARED((NODES, D), jnp.float32),
]
_AGG_SCRATCH += [pltpu.VMEM((KA, D), jnp.float32) for _ in range(NBR)]
_AGG_SCRATCH += [pltpu.SemaphoreType.DMA for _ in range(2 * NBR)]


@functools.partial(
    pl.kernel,
    out_type=jax.ShapeDtypeStruct((NC, NODES, D), jnp.float32),
    mesh=_MESH,
    scratch_types=_AGG_SCRATCH,
    compiler_params=pltpu.CompilerParams(use_tc_tiling_on_sc=False))
def _agg(x_hbm, src_hbm, dst_hbm, zero_hbm, agg_out, idxs_all, idxd_all,
         agg_s, *bufs):
  rows_v = bufs[:NBR]
  gsem = bufs[NBR:2 * NBR]
  ssem = bufs[2 * NBR:3 * NBR]
  cid = lax.axis_index("c")
  sid = lax.axis_index("s")
  wid = sid * NC + cid

  pltpu.sync_copy(src_hbm.at[wid], idxs_all)
  pltpu.sync_copy(dst_hbm.at[wid], idxd_all)

  def zcp(j, _):
    b = sid + j * NS

    @pl.when(b < NBLK)
    def _():
      pltpu.sync_copy(zero_hbm.at[pl.ds(b * BR, BR)],
                      agg_s.at[pl.ds(b * BR, BR)])
    return 0
  lax.fori_loop(0, (NBLK + NS - 1) // NS, zcp, 0)
  plsc.subcore_barrier()

  def g_start(c, b):
    pltpu.async_copy(x_hbm.at[idxs_all.at[c]], rows_v[b], gsem[b])

  def g_wait(b):
    pltpu.make_async_copy(x_hbm.at[idxs_all.at[0]], rows_v[b], gsem[b]).wait()

  def s_start(c, b):
    pltpu.async_copy(rows_v[b], agg_s.at[idxd_all.at[c]], ssem[b], add=True)

  def s_wait(b):
    pltpu.make_async_copy(rows_v[b], agg_s.at[idxd_all.at[0]], ssem[b]).wait()

  for b in range(NBR):
    g_start(b, b)

  def outer(i, _):
    g = i * NBR
    for b in range(NBR):
      c = g + b
      g_wait(b)
      s_start(c, b)
      s_wait(b)

      @pl.when(c + NBR < NCHUNK)
      def _():
        g_start(c + NBR, b)
    return 0
  lax.fori_loop(0, NCHUNK // NBR, outer, 0)

  for t in range(NCHUNK - (NCHUNK // NBR) * NBR):
    c = (NCHUNK // NBR) * NBR + t
    b = c % NBR
    g_wait(b)
    s_start(c, b)
    s_wait(b)
  plsc.subcore_barrier()

  def wout(j, _):
    b = sid + j * NS

    @pl.when(b < NBLK)
    def _():
      r0 = b * BR
      pltpu.sync_copy(agg_s.at[pl.ds(r0, BR)], agg_out.at[cid, pl.ds(r0, BR)])
    return 0
  lax.fori_loop(0, (NBLK + NS - 1) // NS, wout, 0)


DNB = 2
ZREP = 4

_DEC_SCRATCH = [
    pltpu.VMEM((LCHUNK, K), jnp.int32),
    pltpu.VMEM((LCHUNK, K), jnp.int32),
    pltpu.VMEM((LPT,), jnp.float32),
]
_DEC_SCRATCH += [pltpu.VMEM((K, D), jnp.float32) for _ in range(2 * DNB)]
_DEC_SCRATCH += [pltpu.SemaphoreType.DMA for _ in range(DNB)]


@functools.partial(
    pl.kernel,
    out_type=jax.ShapeDtypeStruct((ELABEL,), jnp.float32),
    mesh=_MESH,
    scratch_types=_DEC_SCRATCH,
    compiler_params=pltpu.CompilerParams(use_tc_tiling_on_sc=False))
def _decode(z_hbm, li_hbm, lj_hbm, out_hbm, idxi_all, idxj_all, out_v, *bufs):
  zi_v = bufs[:DNB]
  zj_v = bufs[DNB:2 * DNB]
  gsem = bufs[2 * DNB:]
  cid = lax.axis_index("c")
  sid = lax.axis_index("s")
  wid = sid * NC + cid

  pltpu.sync_copy(li_hbm.at[wid], idxi_all)
  pltpu.sync_copy(lj_hbm.at[wid], idxj_all)

  roff = (wid % ZREP) * NODES

  def shift(i, _):
    for kk in range(K // 16):
      sl = pl.ds(kk * 16, 16)
      off = jnp.full((16,), 1, jnp.int32) * roff
      idxi_all[i, sl] = idxi_all[i, sl] + off
      idxj_all[i, sl] = idxj_all[i, sl] + off
    return 0
  lax.fori_loop(0, LCHUNK, shift, 0)

  lanes = lax.iota(jnp.int32, 16)

  def g_start(c, b):
    pltpu.async_copy(z_hbm.at[idxi_all.at[c]], zi_v[b], gsem[b])
    pltpu.async_copy(z_hbm.at[idxj_all.at[c]], zj_v[b], gsem[b])

  def g_wait(b):
    pltpu.make_async_copy(z_hbm.at[idxi_all.at[0]], zi_v[b], gsem[b]).wait()
    pltpu.make_async_copy(z_hbm.at[idxj_all.at[0]], zj_v[b], gsem[b]).wait()

  for b in range(DNB):
    g_start(b, b)

  def comb(u, v, d, mask):
    perm = jnp.bitwise_xor(lanes, d)
    uu = u + jnp.take_along_axis(u, perm, axis=0)
    vv = v + jnp.take_along_axis(v, perm, axis=0)
    return jnp.where(mask, vv, uu)

  masks = {d: (lanes & d) != 0 for d in (1, 2, 4, 8)}

  def compute_chunk(c, b):
    def group(g16, _):
      row0 = g16 * 16
      t = []
      for rr in range(16):
        r = row0 + rr
        acc = zi_v[b][r, pl.ds(0, 16)] * zj_v[b][r, pl.ds(0, 16)]
        for kk in range(1, D // 16):
          sl = pl.ds(kk * 16, 16)
          acc = acc + zi_v[b][r, sl] * zj_v[b][r, sl]
        t.append(acc)
      for d in (1, 2, 4, 8):
        t = [comb(t[2 * j], t[2 * j + 1], d, masks[d])
             for j in range(len(t) // 2)]
      out_v[pl.ds(c * K + row0, 16)] = t[0]
      return 0
    lax.fori_loop(0, K // 16, group, 0)

  def outer(i, _):
    g0 = i * DNB
    for b in range(DNB):
      c = g0 + b
      g_wait(b)
      compute_chunk(c, b)

      @pl.when(c + DNB < LCHUNK)
      def _():
        g_start(c + DNB, b)
    return 0
  lax.fori_loop(0, (LCHUNK - 1) // DNB, outer, 0)

  g_wait((LCHUNK - 1) % DNB)
  compute_chunk(LCHUNK - 1, (LCHUNK - 1) % DNB)

  pltpu.sync_copy(out_v, out_hbm.at[pl.ds(wid * LPT, LPT)])


def _tc_feat(feat, Wr, b):
  R = 1000

  def body(f_ref, wr_ref, b_ref, o_ref):
    o_ref[...] = lax.dot_general(f_ref[...], wr_ref[...],
                                 (((1,), (1,)), ((), ())),
                                 preferred_element_type=jnp.float32) + b_ref[...]

  return pl.pallas_call(
      body,
      grid=(NODES // R,),
      in_specs=[
          pl.BlockSpec((R, D), lambda i: (i, 0)),
          pl.BlockSpec((D, D), lambda i: (0, 0)),
          pl.BlockSpec((1, D), lambda i: (0, 0)),
      ],
      out_specs=pl.BlockSpec((R, D), lambda i: (i, 0)),
      out_shape=jax.ShapeDtypeStruct((NODES, D), jnp.float32),
  )(feat, Wr, b.reshape(1, D))


def _tc_mix(p, pd, fr, Wl, relu, rep=1):
  R = 1000

  def body(p_ref, pd_ref, fr_ref, wl_ref, o_ref):
    agg = p_ref[0] + p_ref[1]
    deg = pd_ref[0, :, :1] + pd_ref[1, :, :1]
    mean = agg / jnp.maximum(deg, 1.0)
    acc = lax.dot_general(mean, wl_ref[...], (((1,), (1,)), ((), ())),
                          preferred_element_type=jnp.float32) + fr_ref[...]
    if relu:
      acc = jnp.maximum(acc, 0.0)
    if rep == 1:
      o_ref[...] = acc
    else:
      o_ref[...] = jnp.broadcast_to(acc[None], (rep, R, D))

  out_specs = (pl.BlockSpec((R, D), lambda i: (i, 0)) if rep == 1 else
               pl.BlockSpec((rep, R, D), lambda i: (0, i, 0)))
  out_shape = (jax.ShapeDtypeStruct((NODES, D), jnp.float32) if rep == 1 else
               jax.ShapeDtypeStruct((rep, NODES, D), jnp.float32))
  return pl.pallas_call(
      body,
      grid=(NODES // R,),
      in_specs=[
          pl.BlockSpec((NC, R, D), lambda i: (0, i, 0)),
          pl.BlockSpec((NC, R, 16), lambda i: (0, i, 0)),
          pl.BlockSpec((R, D), lambda i: (i, 0)),
          pl.BlockSpec((D, D), lambda i: (0, 0)),
      ],
      out_specs=out_specs,
      out_shape=out_shape,
  )(p, pd, fr, Wl)


def kernel(x, edge_index, edge_label_idx, W1l, b1, W1r, W2l, b2, W2r):
  src = edge_index[0].reshape(NW, NCHUNK, KA)
  dst = edge_index[1].reshape(NW, NCHUNK, KA)
  li = edge_label_idx[0].reshape(NW, LCHUNK, K)
  lj = edge_label_idx[1].reshape(NW, LCHUNK, K)

  zeros = jnp.zeros((NODES, D), jnp.float32)
  pd = _deg(dst)
  p1 = _agg(x, src, dst, zeros)
  xr = _tc_feat(x, W1r, b1)
  h = _tc_mix(p1, pd, xr, W1l, relu=True)
  p2 = _agg(h, src, dst, zeros)
  hr = _tc_feat(h, W2r, b2)
  z = _tc_mix(p2, pd, hr, W2l, relu=False, rep=ZREP)
  return _decode(z.reshape(ZREP * NODES, D), li, lj)
